# baseline (speedup 1.0000x reference)
import functools

import jax
import jax.numpy as jnp
from jax import lax
from jax.experimental import pallas as pl
from jax.experimental.pallas import tpu as pltpu
from jax.experimental.pallas import tpu_sc as plsc

NC = 2
NS = 16
NW = NC * NS
LANES = 16
CHUNK = 128


def _sc_degree(dst2, ew2, n_nodes):
    epw = dst2.shape[1]

    @functools.partial(
        pl.kernel,
        out_type=jax.ShapeDtypeStruct((NW, n_nodes), jnp.float32),
        mesh=plsc.VectorSubcoreMesh(core_axis_name="c", subcore_axis_name="s"),
        scratch_types=[
            pltpu.VMEM((epw,), jnp.int32),
            pltpu.VMEM((epw,), jnp.float32),
            pltpu.VMEM((n_nodes,), jnp.float32),
        ],
        compiler_params=pltpu.CompilerParams(needs_layout_passes=False),
    )
    def deg_kernel(dst_hbm, ew_hbm, out_hbm, dst_v, ew_v, deg_v):
        w = lax.axis_index("c") * NS + lax.axis_index("s")
        pltpu.sync_copy(dst_hbm.at[w], dst_v)
        pltpu.sync_copy(ew_hbm.at[w], ew_v)

        def zero_body(i, carry):
            deg_v[pl.ds(i * LANES, LANES)] = jnp.zeros((LANES,), jnp.float32)
            return carry

        lax.fori_loop(0, n_nodes // LANES, zero_body, 0)

        def acc_body(j, carry):
            d = dst_v[pl.ds(j * LANES, LANES)]
            ww = ew_v[pl.ds(j * LANES, LANES)]
            plsc.addupdate_scatter(deg_v, [d], ww)
            return carry

        lax.fori_loop(0, epw // LANES, acc_body, 0)
        pltpu.sync_copy(deg_v, out_hbm.at[w])

    return deg_kernel(dst2, ew2)


def _sc_coef(src2, dst2, ew2, dinv):
    epw = src2.shape[1]
    n_nodes = dinv.shape[0]

    @functools.partial(
        pl.kernel,
        out_type=jax.ShapeDtypeStruct((NW, epw), jnp.float32),
        mesh=plsc.VectorSubcoreMesh(core_axis_name="c", subcore_axis_name="s"),
        scratch_types=[
            pltpu.VMEM((epw,), jnp.int32),
            pltpu.VMEM((epw,), jnp.int32),
            pltpu.VMEM((epw,), jnp.float32),
            pltpu.VMEM((n_nodes,), jnp.float32),
            pltpu.VMEM((epw,), jnp.float32),
        ],
        compiler_params=pltpu.CompilerParams(needs_layout_passes=False),
    )
    def coef_kernel(src_hbm, dst_hbm, ew_hbm, dinv_hbm, out_hbm,
                    src_v, dst_v, ew_v, dinv_v, c_v):
        w = lax.axis_index("c") * NS + lax.axis_index("s")
        pltpu.sync_copy(src_hbm.at[w], src_v)
        pltpu.sync_copy(dst_hbm.at[w], dst_v)
        pltpu.sync_copy(ew_hbm.at[w], ew_v)
        pltpu.sync_copy(dinv_hbm, dinv_v)

        def body(j, carry):
            sl = pl.ds(j * LANES, LANES)
            cc = (plsc.load_gather(dinv_v, [src_v[sl]]) * ew_v[sl]
                  * plsc.load_gather(dinv_v, [dst_v[sl]]))
            c_v[sl] = cc
            return carry

        lax.fori_loop(0, epw // LANES, body, 0)
        pltpu.sync_copy(c_v, out_hbm.at[w])

    return coef_kernel(src2, dst2, ew2, dinv)


def _sc_aggregate(src3, dst3, c3, h, zeros_nf):
    n, f = h.shape
    nchunk = src3.shape[1]
    step = (n // NS) // 8 * 8
    size = n - step * (NS - 1)
    assert size % 8 == 0 and size >= step

    SBCH = 16
    nsuper, ntail = divmod(nchunk, SBCH)
    supers = [(g, SBCH) for g in range(nsuper)]
    if ntail:
        supers.append((nsuper, ntail))

    @functools.partial(
        pl.kernel,
        out_type=jax.ShapeDtypeStruct((NC, n, f), jnp.float32),
        mesh=plsc.VectorSubcoreMesh(core_axis_name="c", subcore_axis_name="s"),
        scratch_types=[
            pltpu.VMEM((2 * SBCH, CHUNK), jnp.int32),
            pltpu.VMEM((2 * SBCH, CHUNK), jnp.int32),
            pltpu.VMEM((2 * SBCH, CHUNK), jnp.float32),
            pltpu.VMEM((CHUNK, f), jnp.float32),
            pltpu.VMEM((CHUNK, f), jnp.float32),
            pltpu.VMEM_SHARED((n, f), jnp.float32),
            pltpu.SemaphoreType.DMA,
            pltpu.SemaphoreType.DMA,
            pltpu.SemaphoreType.DMA,
        ],
        compiler_params=pltpu.CompilerParams(needs_layout_passes=False),
    )
    def agg_kernel(src_hbm, dst_hbm, c_hbm, h_hbm, z_hbm, out_hbm,
                   src_sb, dst_sb, c_sb, rows0, rows1, acc_sh,
                   gsem0, gsem1, stsem):
        c_ax = lax.axis_index("c")
        s = lax.axis_index("s")
        w = c_ax * NS + s
        rows = (rows0, rows1)
        gsems = (gsem0, gsem1)

        pltpu.sync_copy(z_hbm.at[pl.ds(s * step, size)],
                        acc_sh.at[pl.ds(s * step, size)])

        def stage_start(g, cnt, half):
            base, r0 = g * SBCH, half * SBCH
            return [
                pltpu.async_copy(hbm.at[w].at[pl.ds(base, cnt)],
                                 sb.at[pl.ds(r0, cnt)], stsem)
                for hbm, sb in ((src_hbm, src_sb), (dst_hbm, dst_sb),
                                (c_hbm, c_sb))
            ]

        def gstart(row, buf_id):
            pltpu.async_copy(h_hbm.at[src_sb.at[row]], rows[buf_id],
                             gsems[buf_id])

        def gwait(row, buf_id):
            pltpu.make_async_copy(h_hbm.at[src_sb.at[row]], rows[buf_id],
                                  gsems[buf_id]).wait()

        def scale_scatter(row, buf_id):
            buf = rows[buf_id]
            crow = c_sb.at[row]

            def rowscale(r2, carry):
                for u in range(2):
                    r = r2 * 2 + u
                    cb = plsc.load_gather(
                        crow, [jnp.full((LANES,), r, jnp.int32)])
                    for k in range(f // LANES):
                        sl = pl.ds(k * LANES, LANES)
                        buf.at[r][sl] = buf.at[r][sl] * cb
                return carry

            lax.fori_loop(0, CHUNK // 2, rowscale, 0)
            pltpu.sync_copy(buf, acc_sh.at[dst_sb.at[row]], add=True)

        for d in stage_start(0, supers[0][1], 0):
            d.wait()
        plsc.subcore_barrier()

        pending_stage = None
        for idx, (g, cnt) in enumerate(supers):
            half = idx % 2
            base_row = half * SBCH
            if pending_stage is not None:
                for d in pending_stage:
                    d.wait()
                pending_stage = None
            if idx + 1 < len(supers):
                g2, cnt2 = supers[idx + 1]
                pending_stage = stage_start(g2, cnt2, 1 - half)
            gstart(base_row, 0)
            if cnt % 2 == 0:
                def pair(q, carry, base_row=base_row, cnt=cnt):
                    j0 = 2 * q
                    gstart(base_row + j0 + 1, 1)
                    gwait(base_row + j0, 0)
                    scale_scatter(base_row + j0, 0)

                    @pl.when(j0 + 2 < cnt)
                    def _():
                        gstart(base_row + j0 + 2, 0)

                    gwait(base_row + j0 + 1, 1)
                    scale_scatter(base_row + j0 + 1, 1)
                    return carry

                lax.fori_loop(0, cnt // 2, pair, 0)
            else:
                assert cnt == 1
                gwait(base_row, 0)
                scale_scatter(base_row, 0)

        plsc.subcore_barrier()
        pltpu.sync_copy(acc_sh.at[pl.ds(s * step, size)],
                        out_hbm.at[c_ax].at[pl.ds(s * step, size)])

    return agg_kernel(src3, dst3, c3, h, zeros_nf)


def _tc_first(deg_parts, x, w1):
    n, f = x.shape

    def body(dp_ref, x_ref, w_ref, dinv_ref, h_ref):
        deg = jnp.sum(dp_ref[...], axis=0)
        dinv_ref[...] = jnp.where(deg > 0, lax.rsqrt(deg), 0.0)
        h_ref[...] = jnp.dot(x_ref[...], w_ref[...],
                             preferred_element_type=jnp.float32)

    return pl.pallas_call(
        body,
        out_shape=(
            jax.ShapeDtypeStruct((n,), jnp.float32),
            jax.ShapeDtypeStruct((n, w1.shape[1]), jnp.float32),
        ),
    )(deg_parts, x, w1)


def _tc_mid(acc, b1, g1, be1, w2):
    n, f = acc.shape[1], acc.shape[2]

    def body(acc_ref, b1_ref, g1_ref, be1_ref, w2_ref, h2_ref):
        a = acc_ref[0] + acc_ref[1] + b1_ref[...][None, :]
        mu = jnp.mean(a, axis=0)
        var = jnp.mean((a - mu[None, :]) ** 2, axis=0)
        zn = g1_ref[...][None, :] * (a - mu[None, :]) \
            / jnp.sqrt(var + 1e-5)[None, :] + be1_ref[...][None, :]
        z = jnp.maximum(zn, 0.0)
        h2_ref[...] = jnp.dot(z, w2_ref[...],
                              preferred_element_type=jnp.float32)

    return pl.pallas_call(
        body,
        out_shape=jax.ShapeDtypeStruct((n, w2.shape[1]), jnp.float32),
    )(acc, b1, g1, be1, w2)


def _tc_final(acc, b2):
    n, f = acc.shape[1], acc.shape[2]

    def body(acc_ref, b2_ref, out_ref):
        out_ref[...] = acc_ref[0] + acc_ref[1] + b2_ref[...][None, :]

    return pl.pallas_call(
        body,
        out_shape=jax.ShapeDtypeStruct((n, f), jnp.float32),
    )(acc, b2)


def kernel(x, edge_index, edge_weight, W1, b1, g1, be1, W2, b2):
    n, f = x.shape
    e = edge_weight.shape[0]

    src = edge_index[0].astype(jnp.int32)
    dst = edge_index[1].astype(jnp.int32)
    loop = jnp.arange(n, dtype=jnp.int32)

    e_all = e + n
    per_worker_chunks = -(-e_all // (NW * CHUNK))
    e_pad = NW * per_worker_chunks * CHUNK
    pad = e_pad - e_all

    src_all = jnp.concatenate([src, loop, jnp.zeros((pad,), jnp.int32)])
    dst_all = jnp.concatenate([dst, loop, jnp.zeros((pad,), jnp.int32)])
    ew_all = jnp.concatenate([
        edge_weight.astype(jnp.float32),
        jnp.ones((n,), jnp.float32),
        jnp.zeros((pad,), jnp.float32),
    ])

    def shuffle(a):
        return a.reshape(per_worker_chunks, NW, CHUNK).transpose(1, 0, 2)

    src3 = shuffle(src_all)
    dst3 = shuffle(dst_all)
    epw = per_worker_chunks * CHUNK
    src2 = src3.reshape(NW, epw)
    dst2 = dst3.reshape(NW, epw)
    ew2 = shuffle(ew_all).reshape(NW, epw)

    zeros_nf = jnp.zeros((n, f), jnp.float32)

    deg_parts = _sc_degree(dst2, ew2, n)
    dinv, h1 = _tc_first(deg_parts, x, W1)
    c3 = _sc_coef(src2, dst2, ew2, dinv).reshape(NW, per_worker_chunks, CHUNK)
    acc1 = _sc_aggregate(src3, dst3, c3, h1, zeros_nf)
    h2 = _tc_mid(acc1, b1, g1, be1, W2)
    acc2 = _sc_aggregate(src3, dst3, c3, h2, zeros_nf)
    return _tc_final(acc2, b2)


# speedup vs baseline: 1.1050x; 1.1050x over previous
import functools

import jax
import jax.numpy as jnp
from jax import lax
from jax.experimental import pallas as pl
from jax.experimental.pallas import tpu as pltpu
from jax.experimental.pallas import tpu_sc as plsc

NC = 2
NS = 16
NW = NC * NS
LANES = 16
CHUNK = 96


def _sc_degree(dst2, ew2, n_nodes):
    epw = dst2.shape[1]

    @functools.partial(
        pl.kernel,
        out_type=jax.ShapeDtypeStruct((NW, n_nodes), jnp.float32),
        mesh=plsc.VectorSubcoreMesh(core_axis_name="c", subcore_axis_name="s"),
        scratch_types=[
            pltpu.VMEM((epw,), jnp.int32),
            pltpu.VMEM((epw,), jnp.float32),
            pltpu.VMEM((n_nodes,), jnp.float32),
        ],
        compiler_params=pltpu.CompilerParams(needs_layout_passes=False),
    )
    def deg_kernel(dst_hbm, ew_hbm, out_hbm, dst_v, ew_v, deg_v):
        w = lax.axis_index("c") * NS + lax.axis_index("s")
        pltpu.sync_copy(dst_hbm.at[w], dst_v)
        pltpu.sync_copy(ew_hbm.at[w], ew_v)

        def zero_body(i, carry):
            deg_v[pl.ds(i * LANES, LANES)] = jnp.zeros((LANES,), jnp.float32)
            return carry

        lax.fori_loop(0, n_nodes // LANES, zero_body, 0)

        def acc_body(j, carry):
            d = dst_v[pl.ds(j * LANES, LANES)]
            ww = ew_v[pl.ds(j * LANES, LANES)]
            plsc.addupdate_scatter(deg_v, [d], ww)
            return carry

        lax.fori_loop(0, epw // LANES, acc_body, 0)
        pltpu.sync_copy(deg_v, out_hbm.at[w])

    return deg_kernel(dst2, ew2)


def _sc_coef(src2, dst2, ew2, dinv):
    epw = src2.shape[1]
    n_nodes = dinv.shape[0]

    @functools.partial(
        pl.kernel,
        out_type=jax.ShapeDtypeStruct((NW, epw), jnp.float32),
        mesh=plsc.VectorSubcoreMesh(core_axis_name="c", subcore_axis_name="s"),
        scratch_types=[
            pltpu.VMEM((epw,), jnp.int32),
            pltpu.VMEM((epw,), jnp.int32),
            pltpu.VMEM((epw,), jnp.float32),
            pltpu.VMEM((n_nodes,), jnp.float32),
            pltpu.VMEM((epw,), jnp.float32),
        ],
        compiler_params=pltpu.CompilerParams(needs_layout_passes=False),
    )
    def coef_kernel(src_hbm, dst_hbm, ew_hbm, dinv_hbm, out_hbm,
                    src_v, dst_v, ew_v, dinv_v, c_v):
        w = lax.axis_index("c") * NS + lax.axis_index("s")
        pltpu.sync_copy(src_hbm.at[w], src_v)
        pltpu.sync_copy(dst_hbm.at[w], dst_v)
        pltpu.sync_copy(ew_hbm.at[w], ew_v)
        pltpu.sync_copy(dinv_hbm, dinv_v)

        def body(j, carry):
            sl = pl.ds(j * LANES, LANES)
            cc = (plsc.load_gather(dinv_v, [src_v[sl]]) * ew_v[sl]
                  * plsc.load_gather(dinv_v, [dst_v[sl]]))
            c_v[sl] = cc
            return carry

        lax.fori_loop(0, epw // LANES, body, 0)
        pltpu.sync_copy(c_v, out_hbm.at[w])

    return coef_kernel(src2, dst2, ew2, dinv)


def _sc_aggregate(src3, dst3, c3, h, zeros_nf):
    n, f = h.shape
    nchunk = src3.shape[1]
    step = (n // NS) // 8 * 8
    size = n - step * (NS - 1)
    assert size % 8 == 0 and size >= step

    SBCH = 16
    nsuper, ntail = divmod(nchunk, SBCH)
    supers = [SBCH] * nsuper + ([ntail] if ntail else [])
    rows_of, super_of = [], []
    for idx, cnt in enumerate(supers):
        for j in range(cnt):
            rows_of.append((idx % 2) * SBCH + j)
            super_of.append(idx)
    total = len(rows_of)
    last_chunk_of = [max(t for t in range(total) if super_of[t] == i)
                     for i in range(len(supers))]

    @functools.partial(
        pl.kernel,
        out_type=jax.ShapeDtypeStruct((NC, n, f), jnp.float32),
        mesh=plsc.VectorSubcoreMesh(core_axis_name="c", subcore_axis_name="s"),
        scratch_types=[
            pltpu.VMEM((2 * SBCH, CHUNK), jnp.int32),
            pltpu.VMEM((2 * SBCH, CHUNK), jnp.int32),
            pltpu.VMEM((2 * SBCH, CHUNK), jnp.float32),
            pltpu.VMEM((CHUNK, f), jnp.float32),
            pltpu.VMEM((CHUNK, f), jnp.float32),
            pltpu.VMEM((CHUNK, f), jnp.float32),
            pltpu.VMEM_SHARED((n, f), jnp.float32),
            pltpu.SemaphoreType.DMA,
            pltpu.SemaphoreType.DMA,
            pltpu.SemaphoreType.DMA,
            pltpu.SemaphoreType.DMA,
            pltpu.SemaphoreType.DMA,
            pltpu.SemaphoreType.DMA,
            pltpu.SemaphoreType.DMA,
        ],
        compiler_params=pltpu.CompilerParams(needs_layout_passes=False),
    )
    def agg_kernel(src_hbm, dst_hbm, c_hbm, h_hbm, z_hbm, out_hbm,
                   src_sb, dst_sb, c_sb, rows0, rows1, rows2, acc_sh,
                   gsem0, gsem1, gsem2, ssem0, ssem1, ssem2, stsem):
        c_ax = lax.axis_index("c")
        s = lax.axis_index("s")
        w = c_ax * NS + s
        rows = (rows0, rows1, rows2)
        gsems = (gsem0, gsem1, gsem2)
        ssems = (ssem0, ssem1, ssem2)

        pltpu.sync_copy(z_hbm.at[pl.ds(s * step, size)],
                        acc_sh.at[pl.ds(s * step, size)])

        def stage_start(idx):
            base, r0 = idx * SBCH, (idx % 2) * SBCH
            cnt = supers[idx]
            return [
                pltpu.async_copy(hbm.at[w].at[pl.ds(base, cnt)],
                                 sb.at[pl.ds(r0, cnt)], stsem)
                for hbm, sb in ((src_hbm, src_sb), (dst_hbm, dst_sb),
                                (c_hbm, c_sb))
            ]

        def gstart(t):
            b = t % 3
            pltpu.async_copy(h_hbm.at[src_sb.at[rows_of[t]]], rows[b],
                             gsems[b])

        def gwait(t):
            b = t % 3
            pltpu.make_async_copy(h_hbm.at[src_sb.at[rows_of[t]]], rows[b],
                                  gsems[b]).wait()

        def sscat_start(t):
            b = t % 3
            pltpu.async_copy(rows[b], acc_sh.at[dst_sb.at[rows_of[t]]],
                             ssems[b], add=True)

        def sscat_wait(t):
            b = t % 3
            pltpu.make_async_copy(rows[b], acc_sh.at[dst_sb.at[rows_of[t]]],
                                  ssems[b]).wait()

        def scale(t):
            buf = rows[t % 3]
            crow = c_sb.at[rows_of[t]]

            def rowscale(r2, carry):
                for u in range(2):
                    r = r2 * 2 + u
                    cb = plsc.load_gather(
                        crow, [jnp.full((LANES,), r, jnp.int32)])
                    for k in range(f // LANES):
                        sl = pl.ds(k * LANES, LANES)
                        buf.at[r][sl] = buf.at[r][sl] * cb
                return carry

            lax.fori_loop(0, CHUNK // 2, rowscale, 0)

        for d in stage_start(0):
            d.wait()
        pending = {}
        if len(supers) > 1:
            pending[1] = stage_start(1)
        plsc.subcore_barrier()
        gstart(0)
        if total > 1:
            gstart(1)

        stage_trigger = {last_chunk_of[i] + 2: i + 2
                         for i in range(len(supers) - 2)}
        for t in range(total):
            nxt = t + 2
            if nxt < total and super_of[nxt] in pending:
                for d in pending.pop(super_of[nxt]):
                    d.wait()
            gwait(t)
            scale(t)
            sscat_start(t)
            if t >= 1 and nxt < total:
                sscat_wait(t - 1)
            if nxt < total:
                gstart(nxt)
            if t in stage_trigger:
                idx2 = stage_trigger[t]
                pending[idx2] = stage_start(idx2)

        for t in range(max(0, total - 3), total):
            sscat_wait(t)
        plsc.subcore_barrier()
        pltpu.sync_copy(acc_sh.at[pl.ds(s * step, size)],
                        out_hbm.at[c_ax].at[pl.ds(s * step, size)])

    return agg_kernel(src3, dst3, c3, h, zeros_nf)


def _tc_first(deg_parts, x, w1):
    n, f = x.shape

    def body(dp_ref, x_ref, w_ref, dinv_ref, h_ref):
        deg = jnp.sum(dp_ref[...], axis=0)
        dinv_ref[...] = jnp.where(deg > 0, lax.rsqrt(deg), 0.0)
        h_ref[...] = jnp.dot(x_ref[...], w_ref[...],
                             preferred_element_type=jnp.float32)

    return pl.pallas_call(
        body,
        out_shape=(
            jax.ShapeDtypeStruct((n,), jnp.float32),
            jax.ShapeDtypeStruct((n, w1.shape[1]), jnp.float32),
        ),
    )(deg_parts, x, w1)


def _tc_mid(acc, b1, g1, be1, w2):
    n, f = acc.shape[1], acc.shape[2]

    def body(acc_ref, b1_ref, g1_ref, be1_ref, w2_ref, h2_ref):
        a = acc_ref[0] + acc_ref[1] + b1_ref[...][None, :]
        mu = jnp.mean(a, axis=0)
        var = jnp.mean((a - mu[None, :]) ** 2, axis=0)
        zn = g1_ref[...][None, :] * (a - mu[None, :]) \
            / jnp.sqrt(var + 1e-5)[None, :] + be1_ref[...][None, :]
        z = jnp.maximum(zn, 0.0)
        h2_ref[...] = jnp.dot(z, w2_ref[...],
                              preferred_element_type=jnp.float32)

    return pl.pallas_call(
        body,
        out_shape=jax.ShapeDtypeStruct((n, w2.shape[1]), jnp.float32),
    )(acc, b1, g1, be1, w2)


def _tc_final(acc, b2):
    n, f = acc.shape[1], acc.shape[2]

    def body(acc_ref, b2_ref, out_ref):
        out_ref[...] = acc_ref[0] + acc_ref[1] + b2_ref[...][None, :]

    return pl.pallas_call(
        body,
        out_shape=jax.ShapeDtypeStruct((n, f), jnp.float32),
    )(acc, b2)


def kernel(x, edge_index, edge_weight, W1, b1, g1, be1, W2, b2):
    n, f = x.shape
    e = edge_weight.shape[0]

    src = edge_index[0].astype(jnp.int32)
    dst = edge_index[1].astype(jnp.int32)
    loop = jnp.arange(n, dtype=jnp.int32)

    e_all = e + n
    per_worker_chunks = -(-e_all // (NW * CHUNK))
    e_pad = NW * per_worker_chunks * CHUNK
    pad = e_pad - e_all

    src_all = jnp.concatenate([src, loop, jnp.zeros((pad,), jnp.int32)])
    dst_all = jnp.concatenate([dst, loop, jnp.zeros((pad,), jnp.int32)])
    ew_all = jnp.concatenate([
        edge_weight.astype(jnp.float32),
        jnp.ones((n,), jnp.float32),
        jnp.zeros((pad,), jnp.float32),
    ])

    def shuffle(a):
        return a.reshape(per_worker_chunks, NW, CHUNK).transpose(1, 0, 2)

    src3 = shuffle(src_all)
    dst3 = shuffle(dst_all)
    epw = per_worker_chunks * CHUNK
    src2 = src3.reshape(NW, epw)
    dst2 = dst3.reshape(NW, epw)
    ew2 = shuffle(ew_all).reshape(NW, epw)

    zeros_nf = jnp.zeros((n, f), jnp.float32)

    deg_parts = _sc_degree(dst2, ew2, n)
    dinv, h1 = _tc_first(deg_parts, x, W1)
    c3 = _sc_coef(src2, dst2, ew2, dinv).reshape(NW, per_worker_chunks, CHUNK)
    acc1 = _sc_aggregate(src3, dst3, c3, h1, zeros_nf)
    h2 = _tc_mid(acc1, b1, g1, be1, W2)
    acc2 = _sc_aggregate(src3, dst3, c3, h2, zeros_nf)
    return _tc_final(acc2, b2)
